# baseline (speedup 1.0000x reference)
import functools
import jax
import jax.numpy as jnp
from jax import lax
from jax.experimental import pallas as pl
from jax.experimental.pallas import tpu as pltpu, tpu_sc as plsc

_BATCH = 4096
_IN = 2048
_SH = 256
_PW = 128
_NP = 16
_ND = 64
_BB = 512

_info = plsc.get_sparse_core_info()
_NC, _NS = _info.num_cores, _info.num_subcores
_NW = _NC * _NS
_BPW = _BATCH // _NW


def _sc_body(drug_hbm, tab_hbm, comb_out, idx_v, rows_v, sem):
    wid = lax.axis_index("s") * _NC + lax.axis_index("c")
    base = wid * _BPW
    pltpu.sync_copy(drug_hbm.at[pl.ds(base, _BPW)], idx_v)
    pltpu.async_copy(tab_hbm.at[idx_v], rows_v, sem).wait()
    pltpu.sync_copy(rows_v, comb_out.at[pl.ds(base, _BPW)])


def _sc_gather(drug_indices, table):
    mesh = plsc.VectorSubcoreMesh(core_axis_name="c", subcore_axis_name="s")
    k = pl.kernel(
        _sc_body,
        out_type=jax.ShapeDtypeStruct((_BATCH, 2 * _PW), jnp.float32),
        mesh=mesh,
        scratch_types=[
            pltpu.VMEM((_BPW,), jnp.int32),
            pltpu.VMEM((_BPW, 2 * _PW), jnp.float32),
            pltpu.SemaphoreType.DMA,
        ],
    )
    return k(drug_indices, table)


def _tc_body(x_ref, drug_ref, ws_ref, bs_ref, wp_ref, bp_ref, comb_ref,
             o_ref):
    xb = x_ref[...]
    h = jnp.maximum(
        jnp.dot(xb, ws_ref[...], preferred_element_type=jnp.float32)
        + bs_ref[...], 0.0)
    z = jnp.dot(h, wp_ref[...], preferred_element_type=jnp.float32) + bp_ref[...]
    a = jnp.maximum(z, 0.0)
    drug = drug_ref[...]
    pw = drug % _NP
    wd = comb_ref[:, :_PW]
    bd = comb_ref[:, _PW:_PW + 1]
    colp = jax.lax.broadcasted_iota(jnp.int32, (_BB, _NP * _PW), 1) // _PW
    wd_t = jnp.concatenate([wd] * _NP, axis=1)
    mw = jnp.where(colp == pw, wd_t, 0.0)
    o_ref[...] = jnp.sum(a * mw, axis=1, keepdims=True) + bd


def kernel(x, drug_indices, W_shared, b_shared, W_pw, b_pw, W_drug, b_drug):
    table = jnp.concatenate(
        [W_drug, b_drug.reshape(_ND, 1),
         jnp.zeros((_ND, _PW - 1), jnp.float32)], axis=1)
    comb = _sc_gather(drug_indices, table)

    wp_flat = jnp.transpose(W_pw, (1, 0, 2)).reshape(_SH, _NP * _PW)
    bp_flat = b_pw.reshape(1, _NP * _PW)
    drug2d = drug_indices.reshape(_BATCH, 1)
    bs2d = b_shared.reshape(1, _SH)

    grid = (_BATCH // _BB,)
    out = pl.pallas_call(
        _tc_body,
        grid=grid,
        in_specs=[
            pl.BlockSpec((_BB, _IN), lambda i: (i, 0)),
            pl.BlockSpec((_BB, 1), lambda i: (i, 0)),
            pl.BlockSpec((_IN, _SH), lambda i: (0, 0)),
            pl.BlockSpec((1, _SH), lambda i: (0, 0)),
            pl.BlockSpec((_SH, _NP * _PW), lambda i: (0, 0)),
            pl.BlockSpec((1, _NP * _PW), lambda i: (0, 0)),
            pl.BlockSpec((_BB, 2 * _PW), lambda i: (i, 0)),
        ],
        out_specs=pl.BlockSpec((_BB, 1), lambda i: (i, 0)),
        out_shape=jax.ShapeDtypeStruct((_BATCH, 1), jnp.float32),
    )(x, drug2d, W_shared, bs2d, wp_flat, bp_flat, comb)
    return out.reshape(_BATCH)
